# baseline (speedup 1.0000x reference)
import jax
import jax.numpy as jnp
from jax.experimental import pallas as pl
from jax.experimental.pallas import tpu as pltpu

_CORES = 2
_INNER = 1


def _copy_body(emb_ref, out_ref):
    out_ref[...] = emb_ref[...]


def kernel(x, embedding):
    seq_len = x.shape[1]
    d_model = embedding.shape[-1]
    inner = _INNER
    cores = _CORES
    while seq_len % (cores * inner) != 0 and inner > 1:
        inner //= 2
    if seq_len % (cores * inner) != 0:
        cores, inner = 1, 1
    block = seq_len // (cores * inner)

    spec = pl.BlockSpec(
        (1, block, d_model), lambda i, j, _inner=inner: (0, i * _inner + j, 0)
    )
    return pl.pallas_call(
        _copy_body,
        grid=(cores, inner),
        in_specs=[spec],
        out_specs=spec,
        out_shape=jax.ShapeDtypeStruct((1, seq_len, d_model), embedding.dtype),
        compiler_params=pltpu.CompilerParams(
            dimension_semantics=("arbitrary", "arbitrary"),
        ),
    )(embedding)


# speedup vs baseline: 1.0096x; 1.0096x over previous
import jax
import jax.numpy as jnp
from jax.experimental import pallas as pl
from jax.experimental.pallas import tpu as pltpu

_CORES = 2
_INNER = 1


def _copy_body(emb_ref, out_ref):
    out_ref[...] = emb_ref[...]


def kernel(x, embedding):
    seq_len = x.shape[1]
    d_model = embedding.shape[-1]
    inner = _INNER
    cores = _CORES
    while seq_len % (cores * inner) != 0 and inner > 1:
        inner //= 2
    if seq_len % (cores * inner) != 0:
        cores, inner = 1, 1
    block = seq_len // (cores * inner)

    spec = pl.BlockSpec(
        (1, block, d_model), lambda i, j, _inner=inner: (0, i * _inner + j, 0)
    )
    return pl.pallas_call(
        _copy_body,
        grid=(cores, inner),
        in_specs=[spec],
        out_specs=spec,
        out_shape=jax.ShapeDtypeStruct((1, seq_len, d_model), embedding.dtype),
        compiler_params=pltpu.CompilerParams(
            dimension_semantics=("parallel", "arbitrary"),
        ),
    )(embedding)


# speedup vs baseline: 1.0098x; 1.0002x over previous
import jax
import jax.numpy as jnp
from jax.experimental import pallas as pl
from jax.experimental.pallas import tpu as pltpu

_NUM_BLOCKS = 2


def _copy_body(emb_ref, out_ref):
    out_ref[...] = emb_ref[...]


def kernel(x, embedding):
    seq_len = x.shape[1]
    d_model = embedding.shape[-1]
    nblocks = _NUM_BLOCKS
    while seq_len % nblocks != 0 and nblocks > 1:
        nblocks //= 2
    block = seq_len // nblocks

    spec = pl.BlockSpec((1, block, d_model), lambda i, j: (0, i, 0))
    return pl.pallas_call(
        _copy_body,
        grid=(nblocks, 1),
        in_specs=[spec],
        out_specs=spec,
        out_shape=jax.ShapeDtypeStruct((1, seq_len, d_model), embedding.dtype),
        compiler_params=pltpu.CompilerParams(
            dimension_semantics=("parallel", "arbitrary"),
        ),
    )(embedding)
